# baseline (speedup 1.0000x reference)
import functools

import jax
import jax.numpy as jnp
from jax import lax
from jax.experimental import pallas as pl
from jax.experimental.pallas import tpu as pltpu
from jax.experimental.pallas import tpu_sc as plsc

_B = 16
_S = 2048
_D = 256
_T = 16384

_NC = 2
_NS = 16
_NW = _NC * _NS
_P = 128
_GP = (_B * _S) // _P
_PPB = _S // _P
_NPT = _GP // _NW
_NV = _D // 16


def _body(flat_hbm, cu_hbm, out_hbm, cu_v, idx0, idx1, zbuf, buf0, buf1,
          szbuf, gsem0, gsem1, ssem0, ssem1, zsem, qsem):
    idxs = (idx0, idx1)
    bufs = (buf0, buf1)
    ssems = (ssem0, ssem1)
    del gsem0, gsem1

    cid = lax.axis_index("c")
    sid = lax.axis_index("s")
    wid = sid * _NC + cid

    pltpu.sync_copy(cu_hbm, cu_v)
    lane = lax.iota(jnp.int32, 16)
    cu_lo = cu_v[pl.ds(0, 16)]

    zero16 = jnp.zeros((16,), jnp.float32)

    def zloop(i, carry):
        r = i // _NV
        c = (i % _NV) * 16
        zbuf[r, pl.ds(c, 16)] = zero16
        return carry

    lax.fori_loop(0, _P * _NV, zloop, 0)

    @pl.when(sid == 0)
    def _init_sz():
        pltpu.sync_copy(zbuf, szbuf)

    def piece_info(g):
        b_g = g // _PPB
        s_off = (g % _PPB) * _P
        ivec = jnp.minimum(b_g + lane, _B - 1)
        gathered = cu_lo.at[ivec].get(mode="promise_in_bounds")
        start = gathered[0]
        end = jnp.where(b_g + 1 >= _B, _T, gathered[1])
        lim = jnp.minimum(end - start, _S)
        pv = jnp.clip(lim - s_off, 0, _P)
        return b_g, s_off, start + s_off, pv

    gs = [wid + _NW * j for j in range(_NPT)]
    infos = [piece_info(g) for g in gs]
    pvs = [inf[3] for inf in infos]

    nxt = cu_lo.at[jnp.minimum(lane + 1, _B - 1)].get(
        mode="promise_in_bounds")
    cu_hi = jnp.where(lane == _B - 1, _T, nxt)
    lim_all = jnp.minimum(cu_hi - cu_lo, _S)
    acc = lim_all
    for sh in (1, 2, 4, 8):
        rot = acc.at[(lane + sh) % 16].get(mode="promise_in_bounds")
        acc = acc + rot
    v_rows = acc[0]
    z_rows = _B * _S - v_rows
    x_num = 88 * z_rows - 134 * v_rows
    x_rows = jnp.clip(x_num // 155, 0, z_rows)
    q8 = (8 * x_rows) // jnp.maximum(z_rows, 1)

    def dst(b_g, s_off):
        return out_hbm.at[b_g, pl.ds(s_off, _P), :]

    plsc.subcore_barrier()

    for j in range(_NPT):
        b_g, s_off, _, _ = infos[j]
        is_zero = pvs[j] == 0
        to_stream = (gs[j] % 8) < q8

        @pl.when(jnp.logical_and(is_zero, to_stream))
        def _zs(b_g=b_g, s_off=s_off):
            pltpu.make_async_copy(zbuf, dst(b_g, s_off), qsem).start()

        @pl.when(jnp.logical_and(is_zero, jnp.logical_not(to_stream)))
        def _zd(b_g=b_g, s_off=s_off):
            pltpu.make_async_copy(szbuf, dst(b_g, s_off), zsem).start()

    def fill_idx(i_ref, src):
        for k in range(_P // 16):
            i_ref[pl.ds(k * 16, 16)] = jnp.minimum(src + k * 16 + lane,
                                                   _T - 1)

    b0, so0, _, _ = infos[0]
    fired = [jnp.int32(0), jnp.int32(0)]
    for j in range(_NPT):
        slot = j % 2
        b_g, s_off, src, pv = infos[j]

        @pl.when(jnp.logical_and(pv > 0, fired[slot] > 0))
        def _wait_prev(slot=slot):
            pltpu.make_async_copy(bufs[slot], dst(b0, so0),
                                  ssems[slot]).wait()

        @pl.when(pv > 0)
        def _go(j=j, slot=slot, b_g=b_g, s_off=s_off, src=src, pv=pv):
            fill_idx(idxs[slot], src)
            pltpu.sync_copy(flat_hbm.at[idxs[slot]], bufs[slot])

            @pl.when(pv < _P)
            def _tail():
                def tloop(i, c2):
                    r = pv + i // _NV
                    c = (i % _NV) * 16
                    bufs[slot][r, pl.ds(c, 16)] = zero16
                    return c2

                lax.fori_loop(0, (_P - pv) * _NV, tloop, 0)

            pltpu.make_async_copy(bufs[slot], dst(b_g, s_off),
                                  ssems[slot]).start()

        fired[slot] = jnp.where(pv > 0, jnp.int32(1), fired[slot])

    for slot in range(2):
        @pl.when(fired[slot] > 0)
        def _ds(slot=slot):
            pltpu.make_async_copy(bufs[slot], dst(b0, so0),
                                  ssems[slot]).wait()

    nz_stream = jnp.int32(0)
    nz_spmem = jnp.int32(0)
    for j in range(_NPT):
        is_zero = jnp.where(pvs[j] == 0, 1, 0)
        to_stream = jnp.where((gs[j] % 8) < q8, 1, 0)
        nz_stream = nz_stream + is_zero * to_stream
        nz_spmem = nz_spmem + is_zero * (1 - to_stream)

    def qdrain(i, carry):
        pltpu.make_async_copy(zbuf, dst(b0, so0), qsem).wait()
        return carry

    lax.fori_loop(0, nz_stream, qdrain, 0)

    def zdrain(i, carry):
        pltpu.make_async_copy(szbuf, dst(b0, so0), zsem).wait()
        return carry

    lax.fori_loop(0, nz_spmem, zdrain, 0)


_sc_kernel = functools.partial(
    pl.kernel,
    out_type=jax.ShapeDtypeStruct((_B, _S, _D), jnp.float32),
    mesh=plsc.VectorSubcoreMesh(core_axis_name="c", subcore_axis_name="s"),
    scratch_types=[
        pltpu.VMEM((_B + 1,), jnp.int32),
        pltpu.VMEM((_P,), jnp.int32),
        pltpu.VMEM((_P,), jnp.int32),
        pltpu.VMEM((_P, _D), jnp.float32),
        pltpu.VMEM((_P, _D), jnp.float32),
        pltpu.VMEM((_P, _D), jnp.float32),
        pltpu.VMEM_SHARED((_P, _D), jnp.float32),
        pltpu.SemaphoreType.DMA,
        pltpu.SemaphoreType.DMA,
        pltpu.SemaphoreType.DMA,
        pltpu.SemaphoreType.DMA,
        pltpu.SemaphoreType.DMA,
        pltpu.SemaphoreType.DMA,
    ],
)(_body)


@jax.jit
def kernel(flat_values, cu_seqlens):
    return _sc_kernel(flat_values, cu_seqlens)
